# baseline (speedup 1.0000x reference)
import functools

import jax
import jax.numpy as jnp
from jax import lax
from jax.experimental import pallas as pl
from jax.experimental.pallas import tpu as pltpu
from jax.experimental.pallas import tpu_sc as plsc

N = 10000
E = 320000
H = 200
NT = 32
NPT = 320
NP = NT * NPT
EPT = E // NT
F = 48
NCH = 5
HP = NCH * F
SEGCAP = 1024
LBUF = 64
BK = 2000
REB = 128
KG = 16384

_MESH = plsc.VectorSubcoreMesh(core_axis_name="c", subcore_axis_name="s")


def _wid():
    return lax.axis_index("s") * 2 + lax.axis_index("c")


def _owner(d):
    return ((d >> 6) * 6554) >> 15


def _fill_ref(ref, rows, cols, val):
    v = jnp.full((16,), val, jnp.float32)

    def body(i, _):
        for j in range(cols // 16):
            ref[i, pl.ds(j * 16, 16)] = v
        return 0

    lax.fori_loop(0, rows, body, 0)


@functools.partial(
    pl.kernel,
    out_type=[
        jax.ShapeDtypeStruct((NT * NT * SEGCAP,), jnp.int32),
        jax.ShapeDtypeStruct((NT * NT * SEGCAP,), jnp.int32),
        jax.ShapeDtypeStruct((NT * NT * SEGCAP,), jnp.float32),
        jax.ShapeDtypeStruct((NT * NT,), jnp.int32),
    ],
    mesh=_MESH,
    scratch_types=[
        pltpu.VMEM((BK,), jnp.int32),
        pltpu.VMEM((BK,), jnp.int32),
        pltpu.VMEM((BK,), jnp.float32),
        pltpu.VMEM((NT * LBUF,), jnp.int32),
        pltpu.VMEM((NT * LBUF,), jnp.int32),
        pltpu.VMEM((NT * LBUF,), jnp.float32),
        pltpu.VMEM((NT,), jnp.int32),
        pltpu.SMEM((NT,), jnp.int32),
    ],
    compiler_params=pltpu.CompilerParams(needs_layout_passes=False),
)
def _bucket_kernel(src_h, dst_h, w_h, src_b, dst_b, w_b, hist_h,
                   srcv, dstv, wv, lbs, lbd, lbw, histv, cnt):
    wid = _wid()
    base = pl.multiple_of(wid * EPT, 16)
    lanes = lax.iota(jnp.int32, 16)

    def zcnt(i, _):
        cnt[i] = 0
        return 0
    lax.fori_loop(0, NT, zcnt, 0)

    for blk in range(EPT // BK):
        boff = blk * BK
        pltpu.sync_copy(src_h.at[pl.ds(base + boff, BK)], srcv)
        pltpu.sync_copy(dst_h.at[pl.ds(base + boff, BK)], dstv)
        pltpu.sync_copy(w_h.at[pl.ds(base + boff, BK)], wv)

        def grp_body(g, _):
            sl = pl.ds(g * 16, 16)
            sv = srcv[sl]
            dv = dstv[sl]
            wvv = wv[sl]
            ov = _owner(dv)
            dlv = dv - ov * NPT

            for l in range(16):
                o = ov[l]
                c = cnt[o]
                pos = c & (LBUF - 1)
                tgt = jnp.broadcast_to(o * LBUF + pos, (16,))
                msk = lanes == l
                plsc.store_scatter(lbs, [tgt], sv, mask=msk)
                plsc.store_scatter(lbd, [tgt], dlv, mask=msk)
                plsc.store_scatter(lbw, [tgt], wvv, mask=msk)
                cnt[o] = c + 1

                @pl.when(pos == LBUF - 1)
                def _flush():
                    hoff = pl.multiple_of((o * NT + wid) * SEGCAP + (c - (LBUF - 1)), LBUF)
                    lo = pl.multiple_of(o * LBUF, LBUF)
                    pltpu.sync_copy(lbs.at[pl.ds(lo, LBUF)],
                                    src_b.at[pl.ds(hoff, LBUF)])
                    pltpu.sync_copy(lbd.at[pl.ds(lo, LBUF)],
                                    dst_b.at[pl.ds(hoff, LBUF)])
                    pltpu.sync_copy(lbw.at[pl.ds(lo, LBUF)],
                                    w_b.at[pl.ds(hoff, LBUF)])
            return 0

        lax.fori_loop(0, BK // 16, grp_body, 0)

    def tail(o, _):
        c = cnt[o]
        rem = c & (LBUF - 1)

        @pl.when(rem > 0)
        def _flush():
            hoff = pl.multiple_of((o * NT + wid) * SEGCAP + (c - rem), LBUF)
            lo = pl.multiple_of(o * LBUF, LBUF)
            pltpu.sync_copy(lbs.at[pl.ds(lo, LBUF)],
                            src_b.at[pl.ds(hoff, LBUF)])
            pltpu.sync_copy(lbd.at[pl.ds(lo, LBUF)],
                            dst_b.at[pl.ds(hoff, LBUF)])
            pltpu.sync_copy(lbw.at[pl.ds(lo, LBUF)],
                            w_b.at[pl.ds(hoff, LBUF)])
        return 0

    lax.fori_loop(0, NT, tail, 0)

    for g in range(NT // 16):
        tgt = lanes + g * 16
        for l in range(16):
            o = g * 16 + l
            v = jnp.broadcast_to(cnt[o], (16,))
            plsc.store_scatter(histv, [tgt], v, mask=(lanes == l))
    pltpu.sync_copy(histv, hist_h.at[pl.ds(pl.multiple_of(wid * NT, NT), NT)])


NBMAX = 320


@functools.partial(
    pl.kernel,
    out_type=[
        jax.ShapeDtypeStruct((NCH, NP, F), jnp.float32),
        jax.ShapeDtypeStruct((NCH, NP, F), jnp.float32),
        jax.ShapeDtypeStruct((NCH, NP, F), jnp.float32),
        jax.ShapeDtypeStruct((NCH, NP, F), jnp.float32),
        jax.ShapeDtypeStruct((NT * NPT * 16,), jnp.float32),
    ],
    mesh=_MESH,
    scratch_types=[
        pltpu.VMEM((NPT + 1, F), jnp.float32),
        pltpu.VMEM((NPT + 1, F), jnp.float32),
        pltpu.VMEM((NPT + 1, F), jnp.float32),
        pltpu.VMEM((NPT + 1, F), jnp.float32),
        pltpu.VMEM(((NPT + 1) * 16,), jnp.float32),
        pltpu.VMEM((NPT, F), jnp.float32),
        pltpu.VMEM((NT * 8 + 16,), jnp.int32),
        pltpu.VMEM((NBMAX * 16,), jnp.int32),
        pltpu.VMEM((3, REB), jnp.int32),
        pltpu.VMEM((3, REB), jnp.int32),
        pltpu.VMEM((3, REB), jnp.float32),
        pltpu.VMEM((2, REB), jnp.int32),
        pltpu.VMEM((2, REB), jnp.int32),
        pltpu.VMEM((2, REB, F), jnp.float32),
        pltpu.VMEM((2, REB, 2 * F), jnp.float32),
        pltpu.SemaphoreType.DMA((3,)),
        pltpu.SemaphoreType.DMA((2,)),
        pltpu.SemaphoreType.DMA((2,)),
    ],
    compiler_params=pltpu.CompilerParams(
        needs_layout_passes=False, use_tc_tiling_on_sc=False),
)
def _edge_kernel(a_st, bv_h, ctab, src_b, dst_b, w_b, histt_h,
                 sum_h, sq_h, mx_h, mn_h, cnt_h,
                 accS, accQ, accX, accN, cntv, bvc, htv,
                 desc, recS, recD, recW, idxr, cidxr, arows, crows,
                 semr, semg, semc):
    wid = _wid()
    node0 = pl.multiple_of(wid * NPT, NPT)
    lanes = lax.iota(jnp.int32, 16)
    pltpu.sync_copy(histt_h.at[pl.ds(pl.multiple_of(wid * NT * 8, NT * 8), NT * 8)],
                    htv.at[pl.ds(0, NT * 8)])

    def zc(i, _):
        cntv[pl.ds(i * 16, 16)] = jnp.zeros((16,), jnp.float32)
        return 0
    lax.fori_loop(0, NPT + 1, zc, 0)

    def seg_desc(wseg, nb):
        seglen = htv[pl.ds(pl.multiple_of(wseg * 8, 8), 16)][0]
        soff = pl.multiple_of((wid * NT + wseg) * SEGCAP, REB)
        nblk = (seglen + (REB - 1)) >> 7

        def blk_desc(b, nb2):
            off = soff + b * REB
            val = jnp.minimum(REB, seglen - b * REB)
            tgt = jnp.broadcast_to(nb2 * 16, (16,)) + lanes
            x = jnp.where(lanes == 0, off, val)
            plsc.store_scatter(desc, [tgt], x, mask=(lanes < 2))
            return nb2 + 1

        return lax.fori_loop(0, nblk, blk_desc, nb)

    nbtot = lax.fori_loop(0, NT, seg_desc, 0)

    def rd_desc(b):
        row = desc[pl.ds(pl.multiple_of(b * 16, 16), 16)]
        return row[0], row[1]

    def fire_rec(b, slot):
        off, _ = rd_desc(b)
        off = pl.multiple_of(off, REB)
        pltpu.async_copy(src_b.at[pl.ds(off, REB)], recS.at[slot], semr.at[slot])
        pltpu.async_copy(dst_b.at[pl.ds(off, REB)], recD.at[slot], semr.at[slot])
        pltpu.async_copy(w_b.at[pl.ds(off, REB)], recW.at[slot], semr.at[slot])

    def wait_rec(slot):
        pltpu.make_async_copy(src_b.at[pl.ds(0, REB)], recS.at[slot], semr.at[slot]).wait()
        pltpu.make_async_copy(dst_b.at[pl.ds(0, REB)], recD.at[slot], semr.at[slot]).wait()
        pltpu.make_async_copy(w_b.at[pl.ds(0, REB)], recW.at[slot], semr.at[slot]).wait()

    def pass_body(p, _):
        pltpu.sync_copy(bv_h.at[p, pl.ds(node0, NPT)], bvc)
        _fill_ref(accS, NPT + 1, F, 0.0)
        _fill_ref(accQ, NPT + 1, F, 0.0)
        _fill_ref(accX, NPT + 1, F, -3.0e38)
        _fill_ref(accN, NPT + 1, F, 3.0e38)

        def fire_gather(b, rslot, gslot):
            for i in range(REB // 16):
                sl = pl.ds(i * 16, 16)
                iv = recS[rslot, sl] + p * NP
                idxr[gslot, sl] = jnp.minimum(jnp.maximum(iv, 0), NCH * NP - 1)
                wcv = jnp.minimum(jnp.maximum(recW[rslot, sl], 0.0), 1.0)
                giv = jnp.minimum((wcv * float(KG)).astype(jnp.int32), KG - 1)
                cidxr[gslot, sl] = giv + p * KG
            pltpu.async_copy(a_st.at[idxr.at[gslot]], arows.at[gslot],
                             semg.at[gslot])
            pltpu.async_copy(ctab.at[cidxr.at[gslot]], crows.at[gslot],
                             semc.at[gslot])

        def wait_gather(gslot):
            pltpu.make_async_copy(a_st.at[pl.ds(0, REB)], arows.at[gslot],
                                  semg.at[gslot]).wait()
            pltpu.make_async_copy(ctab.at[pl.ds(0, REB)], crows.at[gslot],
                                  semc.at[gslot]).wait()

        @pl.when(nbtot > 0)
        def _pro():
            fire_rec(0, 0)
            wait_rec(0)
            fire_gather(0, 0, 0)

            @pl.when(nbtot > 1)
            def _pro2():
                fire_rec(1, 1)

        def blk_body(b, _):
            rslot = b - (b // 3) * 3
            gslot = b & 1
            nslot = (b + 1) - ((b + 1) // 3) * 3
            ngslot = (b + 1) & 1

            @pl.when(b + 1 < nbtot)
            def _prefetch():
                wait_rec(nslot)
                fire_gather(b + 1, nslot, ngslot)

                @pl.when(b + 2 < nbtot)
                def _pf2():
                    fire_rec(b + 2, (b + 2) - ((b + 2) // 3) * 3)

            wait_gather(gslot)
            _, valid_n = rd_desc(b)

            def grp_body(g, _):
                gsl = pl.ds(g * 16, 16)
                dlv = recD[rslot, gsl]
                wvv = recW[rslot, gsl]
                gbase = g * 16
                uv = jnp.minimum(jnp.maximum(wvv, 0.0), 1.0) * float(KG)
                iv = jnp.minimum(uv.astype(jnp.int32), KG - 1)
                tv = uv - iv.astype(jnp.float32)

                for l in range(16):
                    ok = gbase + l < valid_n
                    dl = jnp.where(ok, dlv[l], NPT)
                    t = tv[l]
                    e = gbase + l
                    m0 = arows[gslot, e, pl.ds(0, 16)] + bvc[dl, pl.ds(0, 16)]
                    m1 = arows[gslot, e, pl.ds(16, 16)] + bvc[dl, pl.ds(16, 16)]
                    m2 = arows[gslot, e, pl.ds(32, 16)] + bvc[dl, pl.ds(32, 16)]
                    c00 = crows[gslot, e, pl.ds(0, 16)]
                    c01 = crows[gslot, e, pl.ds(16, 16)]
                    c02 = crows[gslot, e, pl.ds(32, 16)]
                    c10 = crows[gslot, e, pl.ds(48, 16)]
                    c11 = crows[gslot, e, pl.ds(64, 16)]
                    c12 = crows[gslot, e, pl.ds(80, 16)]
                    m0 = m0 + c00 + t * (c10 - c00)
                    m1 = m1 + c01 + t * (c11 - c01)
                    m2 = m2 + c02 + t * (c12 - c02)
                    m0 = jnp.maximum(m0, 0.0)
                    m1 = jnp.maximum(m1, 0.0)
                    m2 = jnp.maximum(m2, 0.0)
                    for j, m in ((0, m0), (1, m1), (2, m2)):
                        fsl = pl.ds(j * 16, 16)
                        accS[dl, fsl] = accS[dl, fsl] + m
                        accQ[dl, fsl] = accQ[dl, fsl] + m * m
                        accX[dl, fsl] = jnp.maximum(accX[dl, fsl], m)
                        accN[dl, fsl] = jnp.minimum(accN[dl, fsl], m)
                    csl = pl.ds(pl.multiple_of(dl * 16, 16), 16)
                    cntv[csl] = cntv[csl] + 1.0
                return 0

            lax.fori_loop(0, REB // 16, grp_body, 0)
            return 0

        lax.fori_loop(0, nbtot, blk_body, 0)

        pltpu.sync_copy(accS.at[pl.ds(0, NPT)], sum_h.at[p, pl.ds(node0, NPT)])
        pltpu.sync_copy(accQ.at[pl.ds(0, NPT)], sq_h.at[p, pl.ds(node0, NPT)])
        pltpu.sync_copy(accX.at[pl.ds(0, NPT)], mx_h.at[p, pl.ds(node0, NPT)])
        pltpu.sync_copy(accN.at[pl.ds(0, NPT)], mn_h.at[p, pl.ds(node0, NPT)])
        return 0

    lax.fori_loop(0, NCH, pass_body, 0)
    pltpu.sync_copy(cntv.at[pl.ds(0, NPT * 16)],
                    cnt_h.at[pl.ds(pl.multiple_of(wid * NPT * 16, NPT * 16), NPT * 16)])


def _head_body(feat_ref, w1_ref, b1_ref, w2_ref, b2_ref, w3_ref, b3_ref, out_ref):
    h = jnp.maximum(feat_ref[...] @ w1_ref[...] + b1_ref[...], 0.0)
    h = jnp.maximum(h @ w2_ref[...] + b2_ref[...], 0.0)
    out_ref[...] = h @ w3_ref[...] + b3_ref[...]


def _ln(x, eps=1e-5):
    m = jnp.mean(x, axis=-1, keepdims=True)
    v = jnp.var(x, axis=-1, keepdims=True)
    return (x - m) / jnp.sqrt(v + eps)


def _seg_max(x, ids, n):
    m = jax.ops.segment_max(x, ids, num_segments=n)
    return jnp.where(jnp.isfinite(m), m, 0.0)


def kernel(mol_x, mol_x_feat, mol_total_fea, residue_esm, residue_prot5, residue_edge_index, residue_edge_weight, mol_batch, prot_batch, W_esm, b_esm, W_prot5, b_prot5, W_seq, b_seq, emb_atom, W_af, b_af, W_mol, b_mol, W_mol2, b_mol2, W_msg, b_msg, W_post, b_post, W_c1, b_c1, W_c2, b_c2, W_c3, b_c3):
    relu = jax.nn.relu
    B = mol_total_fea.shape[0]

    residue_ini = jnp.concatenate(
        [relu(residue_prot5 @ W_prot5 + b_prot5), relu(residue_esm @ W_esm + b_esm)], axis=-1)
    residue_x = relu(residue_ini @ W_seq + b_seq)

    W1 = W_msg[0:H]
    W2 = W_msg[H:2 * H]
    W3 = W_msg[2 * H:3 * H]
    A = residue_x @ W1
    Bv = residue_x @ W2 + b_msg

    A_p = jnp.zeros((NP, HP), jnp.float32).at[:N, :H].set(A)
    A_st = A_p.reshape(NP, NCH, F).transpose(1, 0, 2).reshape(NCH * NP, F)
    Bv_p = jnp.zeros((NP, HP), jnp.float32).at[:N, :H].set(Bv)
    Bv_st = Bv_p.reshape(NP, NCH, F).transpose(1, 0, 2)
    grid = jnp.arange(KG + 1, dtype=jnp.float32) / KG
    mu = jnp.linspace(0.0, 1.0, H)
    rbf_g = jnp.exp(-(((grid[:, None] - mu[None, :]) * H) ** 2))
    C_full = jnp.zeros((KG + 1, HP), jnp.float32).at[:, :H].set(rbf_g @ W3)
    C0 = C_full[:KG].reshape(KG, NCH, F)
    C1 = C_full[1:].reshape(KG, NCH, F)
    ctab = jnp.concatenate([C0, C1], axis=-1).transpose(1, 0, 2).reshape(NCH * KG, 2 * F)

    src = residue_edge_index[0].astype(jnp.int32)
    dst = residue_edge_index[1].astype(jnp.int32)
    w = residue_edge_weight.astype(jnp.float32)

    src_b, dst_b, w_b, hist = _bucket_kernel(src, dst, w)
    histt = jnp.broadcast_to(
        hist.reshape(NT, NT).T[:, :, None], (NT, NT, 8)).reshape(-1)
    sum_h, sq_h, mx_h, mn_h, cnt_h = _edge_kernel(
        A_st, Bv_st, ctab, src_b, dst_b, w_b, histt)

    def unstack(x):
        return x.transpose(1, 0, 2).reshape(NP, HP)[:N, :H]

    cnt = cnt_h.reshape(NT * NPT, 16)[:, 0].reshape(NP)[:N] / NCH
    s = unstack(sum_h)
    q = unstack(sq_h)
    mx = unstack(mx_h)
    mn = unstack(mn_h)
    has = (cnt > 0.5)[:, None]
    cntc = jnp.maximum(jnp.round(cnt), 1.0)[:, None]
    mean = s / cntc
    sq = q / cntc
    std = jnp.sqrt(relu(sq - mean * mean) + 1e-5)
    mx = jnp.where(has, mx, 0.0)
    mn = jnp.where(has, mn, 0.0)
    agg = jnp.concatenate([mean, mn, mx, std], axis=-1)
    residue_x2 = relu(agg @ W_post + b_post)

    atom_x = emb_atom[mol_x] + relu(mol_x_feat @ W_af + b_af)
    mol_total = _ln(relu(mol_total_fea @ W_mol + b_mol) @ W_mol2 + b_mol2)
    residue_max = _seg_max(residue_x2, prot_batch, B)
    pc = jnp.maximum(jax.ops.segment_sum(jnp.ones((N,), jnp.float32), prot_batch, num_segments=B), 1.0)[:, None]
    residue_mean = jax.ops.segment_sum(residue_x2, prot_batch, num_segments=B) / pc
    atom_pool = _seg_max(atom_x, mol_batch, B)
    feat = jnp.concatenate([residue_max, residue_mean, atom_pool, mol_total], axis=-1)

    out = pl.pallas_call(
        _head_body,
        out_shape=jax.ShapeDtypeStruct((B, 1), jnp.float32),
    )(feat, W_c1, b_c1, W_c2, b_c2, W_c3, b_c3)
    return out


# speedup vs baseline: 1.6363x; 1.6363x over previous
import functools

import jax
import jax.numpy as jnp
from jax import lax
from jax.experimental import pallas as pl
from jax.experimental.pallas import tpu as pltpu
from jax.experimental.pallas import tpu_sc as plsc

N = 10000
E = 320000
H = 200
NT = 32
NPT = 320
NP = NT * NPT
EPT = E // NT
F = 32
NCH = 7
HP = NCH * F
SEGCAP = 1024
LBUF = 64
BK = 2000
REB = 128
KG = 4096
RCA = 11392
ACCR = NPT + 1

_MESH = plsc.VectorSubcoreMesh(core_axis_name="c", subcore_axis_name="s")


def _wid():
    return lax.axis_index("s") * 2 + lax.axis_index("c")


def _owner(d):
    return ((d >> 6) * 6554) >> 15


def _fill_ref(ref, rows, cols, val):
    v = jnp.full((16,), val, jnp.float32)

    def body(i, _):
        for j in range(cols // 16):
            ref[i, pl.ds(j * 16, 16)] = v
        return 0

    lax.fori_loop(0, rows, body, 0)


@functools.partial(
    pl.kernel,
    out_type=[
        jax.ShapeDtypeStruct((NT * NT * SEGCAP,), jnp.int32),
        jax.ShapeDtypeStruct((NT * NT * SEGCAP,), jnp.int32),
        jax.ShapeDtypeStruct((NT * NT * SEGCAP,), jnp.float32),
        jax.ShapeDtypeStruct((NT * NT,), jnp.int32),
    ],
    mesh=_MESH,
    scratch_types=[
        pltpu.VMEM((BK,), jnp.int32),
        pltpu.VMEM((BK,), jnp.int32),
        pltpu.VMEM((BK,), jnp.float32),
        pltpu.VMEM((NT * LBUF,), jnp.int32),
        pltpu.VMEM((NT * LBUF,), jnp.int32),
        pltpu.VMEM((NT * LBUF,), jnp.float32),
        pltpu.VMEM((NT,), jnp.int32),
        pltpu.SMEM((NT,), jnp.int32),
    ],
    compiler_params=pltpu.CompilerParams(needs_layout_passes=False),
)
def _bucket_kernel(src_h, dst_h, w_h, src_b, dst_b, w_b, hist_h,
                   srcv, dstv, wv, lbs, lbd, lbw, histv, cnt):
    wid = _wid()
    base = pl.multiple_of(wid * EPT, 16)
    lanes = lax.iota(jnp.int32, 16)

    def zcnt(i, _):
        cnt[i] = 0
        return 0
    lax.fori_loop(0, NT, zcnt, 0)

    for blk in range(EPT // BK):
        boff = blk * BK
        pltpu.sync_copy(src_h.at[pl.ds(base + boff, BK)], srcv)
        pltpu.sync_copy(dst_h.at[pl.ds(base + boff, BK)], dstv)
        pltpu.sync_copy(w_h.at[pl.ds(base + boff, BK)], wv)

        def grp_body(g, _):
            sl = pl.ds(g * 16, 16)
            sv = srcv[sl]
            dv = dstv[sl]
            wvv = wv[sl]
            ov = _owner(dv)
            dlv = dv - ov * NPT

            for l in range(16):
                o = ov[l]
                c = cnt[o]
                pos = c & (LBUF - 1)
                tgt = jnp.broadcast_to(o * LBUF + pos, (16,))
                msk = lanes == l
                plsc.store_scatter(lbs, [tgt], sv, mask=msk)
                plsc.store_scatter(lbd, [tgt], dlv, mask=msk)
                plsc.store_scatter(lbw, [tgt], wvv, mask=msk)
                cnt[o] = c + 1

                @pl.when(pos == LBUF - 1)
                def _flush():
                    hoff = pl.multiple_of((o * NT + wid) * SEGCAP + (c - (LBUF - 1)), LBUF)
                    lo = pl.multiple_of(o * LBUF, LBUF)
                    pltpu.sync_copy(lbs.at[pl.ds(lo, LBUF)],
                                    src_b.at[pl.ds(hoff, LBUF)])
                    pltpu.sync_copy(lbd.at[pl.ds(lo, LBUF)],
                                    dst_b.at[pl.ds(hoff, LBUF)])
                    pltpu.sync_copy(lbw.at[pl.ds(lo, LBUF)],
                                    w_b.at[pl.ds(hoff, LBUF)])
            return 0

        lax.fori_loop(0, BK // 16, grp_body, 0)

    def tail(o, _):
        c = cnt[o]
        rem = c & (LBUF - 1)

        @pl.when(rem > 0)
        def _flush():
            hoff = pl.multiple_of((o * NT + wid) * SEGCAP + (c - rem), LBUF)
            lo = pl.multiple_of(o * LBUF, LBUF)
            pltpu.sync_copy(lbs.at[pl.ds(lo, LBUF)],
                            src_b.at[pl.ds(hoff, LBUF)])
            pltpu.sync_copy(lbd.at[pl.ds(lo, LBUF)],
                            dst_b.at[pl.ds(hoff, LBUF)])
            pltpu.sync_copy(lbw.at[pl.ds(lo, LBUF)],
                            w_b.at[pl.ds(hoff, LBUF)])
        return 0

    lax.fori_loop(0, NT, tail, 0)

    for g in range(NT // 16):
        tgt = lanes + g * 16
        for l in range(16):
            o = g * 16 + l
            v = jnp.broadcast_to(cnt[o], (16,))
            plsc.store_scatter(histv, [tgt], v, mask=(lanes == l))
    pltpu.sync_copy(histv, hist_h.at[pl.ds(pl.multiple_of(wid * NT, NT), NT)])


NBMAX = 320
_NEG = -3.0e38
_POS = 3.0e38


@functools.partial(
    pl.kernel,
    out_type=[
        jax.ShapeDtypeStruct((NCH, NP, F), jnp.float32),
        jax.ShapeDtypeStruct((NCH, NP, F), jnp.float32),
        jax.ShapeDtypeStruct((NCH, NP, F), jnp.float32),
        jax.ShapeDtypeStruct((NCH, NP, F), jnp.float32),
        jax.ShapeDtypeStruct((NT * 328,), jnp.int32),
    ],
    mesh=_MESH,
    scratch_types=[
        pltpu.VMEM((ACCR, F), jnp.float32),
        pltpu.VMEM((ACCR, F), jnp.float32),
        pltpu.VMEM((ACCR, F), jnp.float32),
        pltpu.VMEM((ACCR, F), jnp.float32),
        pltpu.VMEM((NPT, F), jnp.float32),
        pltpu.VMEM((NT * 8 + 16,), jnp.int32),
        pltpu.VMEM((NBMAX * 16,), jnp.int32),
        pltpu.VMEM((RCA,), jnp.int32),
        pltpu.VMEM((RCA,), jnp.int32),
        pltpu.VMEM((328,), jnp.int32),
        pltpu.VMEM((3, REB), jnp.int32),
        pltpu.VMEM((3, REB), jnp.int32),
        pltpu.VMEM((3, REB), jnp.float32),
        pltpu.VMEM((2, REB), jnp.int32),
        pltpu.VMEM((2, REB), jnp.int32),
        pltpu.VMEM((2, REB, F), jnp.float32),
        pltpu.VMEM((2, REB, F), jnp.int32),
        pltpu.SemaphoreType.DMA((3,)),
        pltpu.SemaphoreType.DMA((2,)),
        pltpu.SemaphoreType.DMA((2,)),
        pltpu.SMEM((328,), jnp.int32),
    ],
    compiler_params=pltpu.CompilerParams(
        needs_layout_passes=False, use_tc_tiling_on_sc=False),
)
def _edge_kernel(a_st, bv_st, ctab, src_b, dst_b, w_b, histt_h,
                 sum_h, sq_h, mx_h, mn_h, startv_h,
                 accS, accQ, accX, accN, bvc, htv, desc,
                 srcS, cwS, startv, recS, recD, recW, idxr, cidxr,
                 arows, crows, semr, semg, semc, cnt):
    wid = _wid()
    node0 = pl.multiple_of(wid * NPT, NPT)
    lanes = lax.iota(jnp.int32, 16)
    pltpu.sync_copy(histt_h.at[pl.ds(pl.multiple_of(wid * NT * 8, NT * 8), NT * 8)],
                    htv.at[pl.ds(0, NT * 8)])

    def seg_desc(wseg, nb):
        seglen = htv[pl.ds(pl.multiple_of(wseg * 8, 8), 16)][0]
        soff = pl.multiple_of((wid * NT + wseg) * SEGCAP, REB)
        nblk = (seglen + (REB - 1)) >> 7

        def blk_desc(b, nb2):
            off = soff + b * REB
            val = jnp.minimum(REB, seglen - b * REB)
            tgt = jnp.broadcast_to(nb2 * 16, (16,)) + lanes
            x = jnp.where(lanes == 0, off, val)
            plsc.store_scatter(desc, [tgt], x, mask=(lanes < 2))
            return nb2 + 1

        return lax.fori_loop(0, nblk, blk_desc, nb)

    nbtot = lax.fori_loop(0, NT, seg_desc, 0)

    def rd_desc(b):
        row = desc[pl.ds(pl.multiple_of(b * 16, 16), 16)]
        return row[0], row[1]

    def fire_rec(b, slot):
        off, _ = rd_desc(b)
        off = pl.multiple_of(off, REB)
        pltpu.async_copy(src_b.at[pl.ds(off, REB)], recS.at[slot], semr.at[slot])
        pltpu.async_copy(dst_b.at[pl.ds(off, REB)], recD.at[slot], semr.at[slot])
        pltpu.async_copy(w_b.at[pl.ds(off, REB)], recW.at[slot], semr.at[slot])

    def wait_rec(slot):
        pltpu.make_async_copy(src_b.at[pl.ds(0, REB)], recS.at[slot], semr.at[slot]).wait()
        pltpu.make_async_copy(dst_b.at[pl.ds(0, REB)], recD.at[slot], semr.at[slot]).wait()
        pltpu.make_async_copy(w_b.at[pl.ds(0, REB)], recW.at[slot], semr.at[slot]).wait()

    def scan_records(per_group):
        @pl.when(nbtot > 0)
        def _pro():
            fire_rec(0, 0)

            @pl.when(nbtot > 1)
            def _pro2():
                fire_rec(1, 1)

        def blk_body(b, _):
            rslot = b - (b // 3) * 3
            wait_rec(rslot)

            @pl.when(b + 2 < nbtot)
            def _pf():
                fire_rec(b + 2, (b + 2) - ((b + 2) // 3) * 3)

            _, valid_n = rd_desc(b)

            def grp_body(g, _):
                gsl = pl.ds(g * 16, 16)
                sv = recS[rslot, gsl]
                dv = recD[rslot, gsl]
                wv = recW[rslot, gsl]
                dv = jnp.minimum(jnp.maximum(dv, 0), NPT)
                per_group(g, valid_n, sv, dv, wv)
                return 0

            lax.fori_loop(0, REB // 16, grp_body, 0)
            return 0

        lax.fori_loop(0, nbtot, blk_body, 0)

    def zcnt(i, _):
        cnt[i] = 0
        return 0
    lax.fori_loop(0, 328, zcnt, 0)

    def count_lane(g, valid_n, sv, dv, wv):
        for l in range(16):
            ok = (g * 16 + l) < valid_n
            dl = jnp.where(ok, dv[l], NPT)
            cnt[dl] = cnt[dl] + jnp.where(ok, 1, 0)

    scan_records(count_lane)

    def pfx(i, s):
        c = cnt[i]
        cnt[i] = s
        plsc.store_scatter(startv, [jnp.broadcast_to(i, (16,))],
                           jnp.broadcast_to(s, (16,)), mask=(lanes == 0))
        return s + c

    ntot = lax.fori_loop(0, NPT, pfx, 0)
    plsc.store_scatter(startv, [jnp.broadcast_to(NPT, (16,))],
                       jnp.broadcast_to(ntot, (16,)), mask=(lanes == 0))
    ntots = jnp.minimum(ntot, RCA - REB)

    def sort_lane(g, valid_n, sv, dv, wv):
        psv = sv + (dv << 14)
        wc = jnp.minimum(jnp.maximum(wv, 0.0), 1.0) * float(KG)
        gi = jnp.minimum(wc.astype(jnp.int32), KG - 1)
        tq = ((wc - gi.astype(jnp.float32)) * 511.0 + 0.5).astype(jnp.int32)
        pcv = gi + (jnp.minimum(tq, 511) << 14)
        for l in range(16):
            ok = (g * 16 + l) < valid_n
            dl = jnp.where(ok, dv[l], NPT)
            pos = cnt[dl]
            cnt[dl] = pos + jnp.where(ok, 1, 0)
            posc = jnp.minimum(pos, RCA - 1)
            msk = (lanes == l) & jnp.broadcast_to(ok, (16,))
            tgt = jnp.broadcast_to(posc, (16,))
            plsc.store_scatter(srcS, [tgt], psv, mask=msk)
            plsc.store_scatter(cwS, [tgt], pcv, mask=msk)

    scan_records(sort_lane)

    padv = jnp.broadcast_to(NPT << 14, (16,))
    zvi = jnp.zeros((16,), jnp.int32)
    for g in range(REB // 16):
        tgt = jnp.broadcast_to(ntots, (16,)) + g * 16 + lanes
        plsc.store_scatter(srcS, [tgt], padv)
        plsc.store_scatter(cwS, [tgt], zvi)

    nbk = (ntots + (REB - 1)) >> 7

    def pass_body(p, _):
        pltpu.sync_copy(bv_st.at[p, pl.ds(node0, NPT)], bvc)

        def fire_g(b, gslot):
            for i in range(REB // 16):
                sl = pl.ds(i * 16, 16)
                off = pl.ds(pl.multiple_of(b * REB + i * 16, 16), 16)
                pk = srcS[off]
                idxr[gslot, sl] = (pk & 16383) + p * NP
                cw = cwS[off]
                cidxr[gslot, sl] = (cw & 16383) + p * KG
            pltpu.async_copy(a_st.at[idxr.at[gslot]], arows.at[gslot],
                             semg.at[gslot])
            pltpu.async_copy(ctab.at[cidxr.at[gslot]], crows.at[gslot],
                             semc.at[gslot])

        def wait_g(gslot):
            pltpu.make_async_copy(a_st.at[pl.ds(0, REB)], arows.at[gslot],
                                  semg.at[gslot]).wait()
            pltpu.make_async_copy(ctab.at[pl.ds(0, REB)], crows.at[gslot],
                                  semc.at[gslot]).wait()

        @pl.when(nbk > 0)
        def _pro():
            fire_g(0, 0)

        zf = jnp.zeros((16,), jnp.float32)
        ngv = jnp.full((16,), _NEG, jnp.float32)
        pzv = jnp.full((16,), _POS, jnp.float32)
        init = (zf, zf, zf, zf, ngv, ngv, pzv, pzv, zf, zf, NPT)

        def flush(sS0, sS1, sQ0, sQ1, sX0, sX1, sN0, sN1, dlc):
            accS[dlc, pl.ds(0, 16)] = sS0
            accS[dlc, pl.ds(16, 16)] = sS1
            accQ[dlc, pl.ds(0, 16)] = sQ0
            accQ[dlc, pl.ds(16, 16)] = sQ1
            accX[dlc, pl.ds(0, 16)] = sX0
            accX[dlc, pl.ds(16, 16)] = sX1
            accN[dlc, pl.ds(0, 16)] = sN0
            accN[dlc, pl.ds(16, 16)] = sN1

        def blk_body(b, carry):
            gslot = b & 1

            @pl.when(b + 1 < nbk)
            def _pf():
                fire_g(b + 1, (b + 1) & 1)

            wait_g(gslot)

            def grp_body(g, c2):
                off = pl.ds(pl.multiple_of(b * REB + g * 16, 16), 16)
                pkv = srcS[off]
                dlv = pkv >> 14
                cwv = cwS[off]
                tv = (cwv >> 14).astype(jnp.float32) * (1.0 / 511.0)
                (sS0, sS1, sQ0, sQ1, sX0, sX1, sN0, sN1, b0r, b1r, dlc) = c2

                for l in range(16):
                    dl = dlv[l]
                    t = tv[l]
                    changed = dl != dlc

                    def fl(sS0, sS1, sQ0, sQ1, sX0, sX1, sN0, sN1,
                           b0r, b1r, dlc, dl=dl):
                        flush(sS0, sS1, sQ0, sQ1, sX0, sX1, sN0, sN1, dlc)
                        nb0 = bvc[dl, pl.ds(0, 16)]
                        nb1 = bvc[dl, pl.ds(16, 16)]
                        return (zf, zf, zf, zf, ngv, ngv, pzv, pzv,
                                nb0, nb1, dl)

                    def kp(sS0, sS1, sQ0, sQ1, sX0, sX1, sN0, sN1,
                           b0r, b1r, dlc):
                        return (sS0, sS1, sQ0, sQ1, sX0, sX1, sN0, sN1,
                                b0r, b1r, dlc)

                    (sS0, sS1, sQ0, sQ1, sX0, sX1, sN0, sN1, b0r, b1r,
                     dlc) = lax.cond(changed, fl, kp, sS0, sS1, sQ0, sQ1,
                                     sX0, sX1, sN0, sN1, b0r, b1r, dlc)

                    e = g * 16 + l
                    a0 = arows[gslot, e, pl.ds(0, 16)]
                    a1 = arows[gslot, e, pl.ds(16, 16)]
                    cx0 = crows[gslot, e, pl.ds(0, 16)]
                    cx1 = crows[gslot, e, pl.ds(16, 16)]
                    c0e = plsc.bitcast(cx0 << 16, jnp.float32)
                    c0o = plsc.bitcast(cx0 & (-65536), jnp.float32)
                    c1e = plsc.bitcast(cx1 << 16, jnp.float32)
                    c1o = plsc.bitcast(cx1 & (-65536), jnp.float32)
                    m0 = a0 + b0r + c0e + t * (c1e - c0e)
                    m1 = a1 + b1r + c0o + t * (c1o - c0o)
                    m0 = jnp.maximum(m0, 0.0)
                    m1 = jnp.maximum(m1, 0.0)
                    sS0 = sS0 + m0
                    sS1 = sS1 + m1
                    sQ0 = sQ0 + m0 * m0
                    sQ1 = sQ1 + m1 * m1
                    sX0 = jnp.maximum(sX0, m0)
                    sX1 = jnp.maximum(sX1, m1)
                    sN0 = jnp.minimum(sN0, m0)
                    sN1 = jnp.minimum(sN1, m1)

                return (sS0, sS1, sQ0, sQ1, sX0, sX1, sN0, sN1, b0r, b1r, dlc)

            return lax.fori_loop(0, REB // 16, grp_body, carry)

        fin = lax.fori_loop(0, nbk, blk_body, init)
        (sS0, sS1, sQ0, sQ1, sX0, sX1, sN0, sN1, _, _, dlc) = fin
        flush(sS0, sS1, sQ0, sQ1, sX0, sX1, sN0, sN1, dlc)

        pltpu.sync_copy(accS.at[pl.ds(0, NPT)], sum_h.at[p, pl.ds(node0, NPT)])
        pltpu.sync_copy(accQ.at[pl.ds(0, NPT)], sq_h.at[p, pl.ds(node0, NPT)])
        pltpu.sync_copy(accX.at[pl.ds(0, NPT)], mx_h.at[p, pl.ds(node0, NPT)])
        pltpu.sync_copy(accN.at[pl.ds(0, NPT)], mn_h.at[p, pl.ds(node0, NPT)])
        return 0

    lax.fori_loop(0, NCH, pass_body, 0)
    pltpu.sync_copy(startv.at[pl.ds(0, 328)],
                    startv_h.at[pl.ds(pl.multiple_of(wid * 328, 8), 328)])


def _head_body(feat_ref, w1_ref, b1_ref, w2_ref, b2_ref, w3_ref, b3_ref, out_ref):
    h = jnp.maximum(feat_ref[...] @ w1_ref[...] + b1_ref[...], 0.0)
    h = jnp.maximum(h @ w2_ref[...] + b2_ref[...], 0.0)
    out_ref[...] = h @ w3_ref[...] + b3_ref[...]


def _ln(x, eps=1e-5):
    m = jnp.mean(x, axis=-1, keepdims=True)
    v = jnp.var(x, axis=-1, keepdims=True)
    return (x - m) / jnp.sqrt(v + eps)


def _seg_max(x, ids, n):
    m = jax.ops.segment_max(x, ids, num_segments=n)
    return jnp.where(jnp.isfinite(m), m, 0.0)


def kernel(mol_x, mol_x_feat, mol_total_fea, residue_esm, residue_prot5, residue_edge_index, residue_edge_weight, mol_batch, prot_batch, W_esm, b_esm, W_prot5, b_prot5, W_seq, b_seq, emb_atom, W_af, b_af, W_mol, b_mol, W_mol2, b_mol2, W_msg, b_msg, W_post, b_post, W_c1, b_c1, W_c2, b_c2, W_c3, b_c3):
    relu = jax.nn.relu
    B = mol_total_fea.shape[0]

    residue_ini = jnp.concatenate(
        [relu(residue_prot5 @ W_prot5 + b_prot5), relu(residue_esm @ W_esm + b_esm)], axis=-1)
    residue_x = relu(residue_ini @ W_seq + b_seq)

    W1 = W_msg[0:H]
    W2 = W_msg[H:2 * H]
    W3 = W_msg[2 * H:3 * H]
    A = residue_x @ W1
    Bv = residue_x @ W2 + b_msg

    perm_l = []
    for c in range(NCH):
        perm_l += [c * F + j for j in range(0, F, 2)]
        perm_l += [c * F + j for j in range(1, F, 2)]
    perm = jnp.array(perm_l, jnp.int32)
    inv = jnp.array([perm_l.index(i) for i in range(HP)], jnp.int32)

    A_p = jnp.zeros((NP, HP), jnp.float32).at[:N, :H].set(A)[:, perm]
    A_st = A_p.reshape(NP, NCH, F).transpose(1, 0, 2).reshape(NCH * NP, F)
    Bv_p = jnp.zeros((NP, HP), jnp.float32).at[:N, :H].set(Bv)[:, perm]
    Bv_st = Bv_p.reshape(NP, NCH, F).transpose(1, 0, 2)

    grid = jnp.arange(KG + 1, dtype=jnp.float32) / KG
    mu = jnp.linspace(0.0, 1.0, H)
    rbf_g = jnp.exp(-(((grid[:, None] - mu[None, :]) * H) ** 2))
    C_full = jnp.zeros((KG + 1, HP), jnp.float32).at[:, :H].set(rbf_g @ W3)
    cb = jax.lax.bitcast_convert_type(
        C_full.astype(jnp.bfloat16), jnp.uint16).astype(jnp.uint32)
    cpk = jax.lax.bitcast_convert_type(
        cb[:, 0::2] | (cb[:, 1::2] << 16), jnp.int32)
    P0 = cpk[:KG].reshape(KG, NCH, F // 2)
    P1 = cpk[1:].reshape(KG, NCH, F // 2)
    ctab = jnp.concatenate([P0, P1], axis=-1).transpose(1, 0, 2).reshape(NCH * KG, F)

    src = residue_edge_index[0].astype(jnp.int32)
    dst = residue_edge_index[1].astype(jnp.int32)
    w = residue_edge_weight.astype(jnp.float32)

    src_b, dst_b, w_b, hist = _bucket_kernel(src, dst, w)
    histt = jnp.broadcast_to(
        hist.reshape(NT, NT).T[:, :, None], (NT, NT, 8)).reshape(-1)
    sum_h, sq_h, mx_h, mn_h, startv_h = _edge_kernel(
        A_st, Bv_st, ctab, src_b, dst_b, w_b, histt)

    def unstack(x):
        return x.transpose(1, 0, 2).reshape(NP, HP)[:, inv][:N, :H]

    sh = startv_h.reshape(NT, 328)
    cnt = (sh[:, 1:NPT + 1] - sh[:, :NPT]).reshape(NP)[:N].astype(jnp.float32)
    s = unstack(sum_h)
    q = unstack(sq_h)
    mx = unstack(mx_h)
    mn = unstack(mn_h)
    has = (cnt > 0.5)[:, None]
    cntc = jnp.maximum(cnt, 1.0)[:, None]
    mean = jnp.where(has, s / cntc, 0.0)
    sq = jnp.where(has, q / cntc, 0.0)
    std = jnp.sqrt(relu(sq - mean * mean) + 1e-5)
    mx = jnp.where(has, mx, 0.0)
    mn = jnp.where(has, mn, 0.0)
    agg = jnp.concatenate([mean, mn, mx, std], axis=-1)
    residue_x2 = relu(agg @ W_post + b_post)

    atom_x = emb_atom[mol_x] + relu(mol_x_feat @ W_af + b_af)
    mol_total = _ln(relu(mol_total_fea @ W_mol + b_mol) @ W_mol2 + b_mol2)
    residue_max = _seg_max(residue_x2, prot_batch, B)
    pc = jnp.maximum(jax.ops.segment_sum(jnp.ones((N,), jnp.float32), prot_batch, num_segments=B), 1.0)[:, None]
    residue_mean = jax.ops.segment_sum(residue_x2, prot_batch, num_segments=B) / pc
    atom_pool = _seg_max(atom_x, mol_batch, B)
    feat = jnp.concatenate([residue_max, residue_mean, atom_pool, mol_total], axis=-1)

    out = pl.pallas_call(
        _head_body,
        out_shape=jax.ShapeDtypeStruct((B, 1), jnp.float32),
    )(feat, W_c1, b_c1, W_c2, b_c2, W_c3, b_c3)
    return out


# speedup vs baseline: 1.7505x; 1.0698x over previous
import functools

import jax
import jax.numpy as jnp
from jax import lax
from jax.experimental import pallas as pl
from jax.experimental.pallas import tpu as pltpu
from jax.experimental.pallas import tpu_sc as plsc

N = 10000
E = 320000
H = 200
NT = 32
NPT = 320
NP = NT * NPT
EPT = E // NT
F = 32
NCH = 7
HP = NCH * F
SEGCAP = 1024
LBUF = 64
BK = 2000
REB = 128
KG = 4096
RCA = 11392
ACCR = NPT + 1

_MESH = plsc.VectorSubcoreMesh(core_axis_name="c", subcore_axis_name="s")


def _wid():
    return lax.axis_index("s") * 2 + lax.axis_index("c")


def _owner(d):
    return ((d >> 6) * 6554) >> 15


def _fill_ref(ref, rows, cols, val):
    v = jnp.full((16,), val, jnp.float32)

    def body(i, _):
        for j in range(cols // 16):
            ref[i, pl.ds(j * 16, 16)] = v
        return 0

    lax.fori_loop(0, rows, body, 0)


@functools.partial(
    pl.kernel,
    out_type=[
        jax.ShapeDtypeStruct((NT * NT * SEGCAP,), jnp.int32),
        jax.ShapeDtypeStruct((NT * NT * SEGCAP,), jnp.int32),
        jax.ShapeDtypeStruct((NT * NT * SEGCAP,), jnp.float32),
        jax.ShapeDtypeStruct((NT * NT,), jnp.int32),
    ],
    mesh=_MESH,
    scratch_types=[
        pltpu.VMEM((BK,), jnp.int32),
        pltpu.VMEM((BK,), jnp.int32),
        pltpu.VMEM((BK,), jnp.float32),
        pltpu.VMEM((NT * LBUF,), jnp.int32),
        pltpu.VMEM((NT * LBUF,), jnp.int32),
        pltpu.VMEM((NT * LBUF,), jnp.float32),
        pltpu.VMEM((NT,), jnp.int32),
        pltpu.SMEM((NT,), jnp.int32),
    ],
    compiler_params=pltpu.CompilerParams(needs_layout_passes=False),
)
def _bucket_kernel(src_h, dst_h, w_h, src_b, dst_b, w_b, hist_h,
                   srcv, dstv, wv, lbs, lbd, lbw, histv, cnt):
    wid = _wid()
    base = pl.multiple_of(wid * EPT, 16)
    lanes = lax.iota(jnp.int32, 16)

    def zcnt(i, _):
        cnt[i] = 0
        return 0
    lax.fori_loop(0, NT, zcnt, 0)

    for blk in range(EPT // BK):
        boff = blk * BK
        pltpu.sync_copy(src_h.at[pl.ds(base + boff, BK)], srcv)
        pltpu.sync_copy(dst_h.at[pl.ds(base + boff, BK)], dstv)
        pltpu.sync_copy(w_h.at[pl.ds(base + boff, BK)], wv)

        def grp_body(g, _):
            sl = pl.ds(g * 16, 16)
            sv = srcv[sl]
            dv = dstv[sl]
            wvv = wv[sl]
            ov = _owner(dv)
            dlv = dv - ov * NPT

            for l in range(16):
                o = ov[l]
                c = cnt[o]
                pos = c & (LBUF - 1)
                tgt = jnp.broadcast_to(o * LBUF + pos, (16,))
                msk = lanes == l
                plsc.store_scatter(lbs, [tgt], sv, mask=msk)
                plsc.store_scatter(lbd, [tgt], dlv, mask=msk)
                plsc.store_scatter(lbw, [tgt], wvv, mask=msk)
                cnt[o] = c + 1

                @pl.when(pos == LBUF - 1)
                def _flush():
                    hoff = pl.multiple_of((o * NT + wid) * SEGCAP + (c - (LBUF - 1)), LBUF)
                    lo = pl.multiple_of(o * LBUF, LBUF)
                    pltpu.sync_copy(lbs.at[pl.ds(lo, LBUF)],
                                    src_b.at[pl.ds(hoff, LBUF)])
                    pltpu.sync_copy(lbd.at[pl.ds(lo, LBUF)],
                                    dst_b.at[pl.ds(hoff, LBUF)])
                    pltpu.sync_copy(lbw.at[pl.ds(lo, LBUF)],
                                    w_b.at[pl.ds(hoff, LBUF)])
            return 0

        lax.fori_loop(0, BK // 16, grp_body, 0)

    def tail(o, _):
        c = cnt[o]
        rem = c & (LBUF - 1)

        @pl.when(rem > 0)
        def _flush():
            hoff = pl.multiple_of((o * NT + wid) * SEGCAP + (c - rem), LBUF)
            lo = pl.multiple_of(o * LBUF, LBUF)
            pltpu.sync_copy(lbs.at[pl.ds(lo, LBUF)],
                            src_b.at[pl.ds(hoff, LBUF)])
            pltpu.sync_copy(lbd.at[pl.ds(lo, LBUF)],
                            dst_b.at[pl.ds(hoff, LBUF)])
            pltpu.sync_copy(lbw.at[pl.ds(lo, LBUF)],
                            w_b.at[pl.ds(hoff, LBUF)])
        return 0

    lax.fori_loop(0, NT, tail, 0)

    for g in range(NT // 16):
        tgt = lanes + g * 16
        for l in range(16):
            o = g * 16 + l
            v = jnp.broadcast_to(cnt[o], (16,))
            plsc.store_scatter(histv, [tgt], v, mask=(lanes == l))
    pltpu.sync_copy(histv, hist_h.at[pl.ds(pl.multiple_of(wid * NT, NT), NT)])


NBMAX = 320
_NEG = -3.0e38
_POS = 3.0e38


@functools.partial(
    pl.kernel,
    out_type=[
        jax.ShapeDtypeStruct((NCH, NP, F), jnp.float32),
        jax.ShapeDtypeStruct((NCH, NP, F), jnp.float32),
        jax.ShapeDtypeStruct((NCH, NP, F), jnp.float32),
        jax.ShapeDtypeStruct((NCH, NP, F), jnp.float32),
        jax.ShapeDtypeStruct((NT * 328,), jnp.int32),
    ],
    mesh=_MESH,
    scratch_types=[
        pltpu.VMEM((ACCR, F), jnp.float32),
        pltpu.VMEM((ACCR, F), jnp.float32),
        pltpu.VMEM((ACCR, F), jnp.float32),
        pltpu.VMEM((ACCR, F), jnp.float32),
        pltpu.VMEM((NPT, F), jnp.float32),
        pltpu.VMEM((NT * 8 + 16,), jnp.int32),
        pltpu.VMEM((NBMAX * 16,), jnp.int32),
        pltpu.VMEM((RCA,), jnp.int32),
        pltpu.VMEM((RCA,), jnp.int32),
        pltpu.VMEM((328,), jnp.int32),
        pltpu.VMEM((3, REB), jnp.int32),
        pltpu.VMEM((3, REB), jnp.int32),
        pltpu.VMEM((3, REB), jnp.float32),
        pltpu.VMEM((2, REB), jnp.int32),
        pltpu.VMEM((2, REB), jnp.int32),
        pltpu.VMEM((2, REB, F), jnp.float32),
        pltpu.VMEM((2, REB, F), jnp.int32),
        pltpu.SemaphoreType.DMA((3,)),
        pltpu.SemaphoreType.DMA((2,)),
        pltpu.SemaphoreType.DMA((2,)),
        pltpu.SMEM((328,), jnp.int32),
    ],
    compiler_params=pltpu.CompilerParams(
        needs_layout_passes=False, use_tc_tiling_on_sc=False),
)
def _edge_kernel(a_st, bv_st, ctab, src_b, dst_b, w_b, histt_h,
                 sum_h, sq_h, mx_h, mn_h, startv_h,
                 accS, accQ, accX, accN, bvc, htv, desc,
                 srcS, cwS, startv, recS, recD, recW, idxr, cidxr,
                 arows, crows, semr, semg, semc, cnt):
    wid = _wid()
    node0 = pl.multiple_of(wid * NPT, NPT)
    lanes = lax.iota(jnp.int32, 16)
    pltpu.sync_copy(histt_h.at[pl.ds(pl.multiple_of(wid * NT * 8, NT * 8), NT * 8)],
                    htv.at[pl.ds(0, NT * 8)])

    def seg_desc(wseg, nb):
        seglen = htv[pl.ds(pl.multiple_of(wseg * 8, 8), 16)][0]
        soff = pl.multiple_of((wid * NT + wseg) * SEGCAP, REB)
        nblk = (seglen + (REB - 1)) >> 7

        def blk_desc(b, nb2):
            off = soff + b * REB
            val = jnp.minimum(REB, seglen - b * REB)
            tgt = jnp.broadcast_to(nb2 * 16, (16,)) + lanes
            x = jnp.where(lanes == 0, off, val)
            plsc.store_scatter(desc, [tgt], x, mask=(lanes < 2))
            return nb2 + 1

        return lax.fori_loop(0, nblk, blk_desc, nb)

    nbtot = lax.fori_loop(0, NT, seg_desc, 0)

    def rd_desc(b):
        row = desc[pl.ds(pl.multiple_of(b * 16, 16), 16)]
        return row[0], row[1]

    def fire_rec(b, slot):
        off, _ = rd_desc(b)
        off = pl.multiple_of(off, REB)
        pltpu.async_copy(src_b.at[pl.ds(off, REB)], recS.at[slot], semr.at[slot])
        pltpu.async_copy(dst_b.at[pl.ds(off, REB)], recD.at[slot], semr.at[slot])
        pltpu.async_copy(w_b.at[pl.ds(off, REB)], recW.at[slot], semr.at[slot])

    def wait_rec(slot):
        pltpu.make_async_copy(src_b.at[pl.ds(0, REB)], recS.at[slot], semr.at[slot]).wait()
        pltpu.make_async_copy(dst_b.at[pl.ds(0, REB)], recD.at[slot], semr.at[slot]).wait()
        pltpu.make_async_copy(w_b.at[pl.ds(0, REB)], recW.at[slot], semr.at[slot]).wait()

    def scan_records(per_group):
        @pl.when(nbtot > 0)
        def _pro():
            fire_rec(0, 0)

            @pl.when(nbtot > 1)
            def _pro2():
                fire_rec(1, 1)

        def blk_body(b, _):
            rslot = b - (b // 3) * 3
            wait_rec(rslot)

            @pl.when(b + 2 < nbtot)
            def _pf():
                fire_rec(b + 2, (b + 2) - ((b + 2) // 3) * 3)

            _, valid_n = rd_desc(b)

            def grp_body(g, _):
                gsl = pl.ds(g * 16, 16)
                sv = recS[rslot, gsl]
                dv = recD[rslot, gsl]
                wv = recW[rslot, gsl]
                dv = jnp.minimum(jnp.maximum(dv, 0), NPT)
                per_group(g, valid_n, sv, dv, wv)
                return 0

            lax.fori_loop(0, REB // 16, grp_body, 0)
            return 0

        lax.fori_loop(0, nbtot, blk_body, 0)

    def zcnt(i, _):
        cnt[i] = 0
        return 0
    lax.fori_loop(0, 328, zcnt, 0)

    def count_lane(g, valid_n, sv, dv, wv):
        for l in range(16):
            ok = (g * 16 + l) < valid_n
            dl = jnp.where(ok, dv[l], NPT)
            cnt[dl] = cnt[dl] + jnp.where(ok, 1, 0)

    scan_records(count_lane)

    def pfx(i, s):
        c = cnt[i]
        cnt[i] = s
        plsc.store_scatter(startv, [jnp.broadcast_to(i, (16,))],
                           jnp.broadcast_to(s, (16,)), mask=(lanes == 0))
        return s + c

    ntot = lax.fori_loop(0, NPT, pfx, 0)
    plsc.store_scatter(startv, [jnp.broadcast_to(NPT, (16,))],
                       jnp.broadcast_to(ntot, (16,)), mask=(lanes == 0))
    ntots = jnp.minimum(ntot, RCA - REB)

    def sort_lane(g, valid_n, sv, dv, wv):
        psv = sv + (dv << 14)
        wc = jnp.minimum(jnp.maximum(wv, 0.0), 1.0) * float(KG)
        gi = jnp.minimum(wc.astype(jnp.int32), KG - 1)
        tq = ((wc - gi.astype(jnp.float32)) * 511.0 + 0.5).astype(jnp.int32)
        pcv = gi + (jnp.minimum(tq, 511) << 14)
        for l in range(16):
            ok = (g * 16 + l) < valid_n
            dl = jnp.where(ok, dv[l], NPT)
            pos = cnt[dl]
            cnt[dl] = pos + jnp.where(ok, 1, 0)
            posc = jnp.minimum(pos, RCA - 1)
            msk = (lanes == l) & jnp.broadcast_to(ok, (16,))
            tgt = jnp.broadcast_to(posc, (16,))
            plsc.store_scatter(srcS, [tgt], psv, mask=msk)
            plsc.store_scatter(cwS, [tgt], pcv, mask=msk)

    scan_records(sort_lane)

    padv = jnp.broadcast_to(NPT << 14, (16,))
    zvi = jnp.zeros((16,), jnp.int32)
    for g in range(REB // 16):
        tgt = jnp.broadcast_to(ntots, (16,)) + g * 16 + lanes
        plsc.store_scatter(srcS, [tgt], padv)
        plsc.store_scatter(cwS, [tgt], zvi)

    nbk = (ntots + (REB - 1)) >> 7

    def pass_body(p, _):
        pltpu.sync_copy(bv_st.at[p, pl.ds(node0, NPT)], bvc)

        def fire_g(b, gslot):
            for i in range(REB // 16):
                sl = pl.ds(i * 16, 16)
                off = pl.ds(pl.multiple_of(b * REB + i * 16, 16), 16)
                pk = srcS[off]
                idxr[gslot, sl] = (pk & 16383) + p * NP
                cw = cwS[off]
                cidxr[gslot, sl] = (cw & 16383) + p * KG
            pltpu.async_copy(a_st.at[idxr.at[gslot]], arows.at[gslot],
                             semg.at[gslot])
            pltpu.async_copy(ctab.at[cidxr.at[gslot]], crows.at[gslot],
                             semc.at[gslot])

        def wait_g(gslot):
            pltpu.make_async_copy(a_st.at[pl.ds(0, REB)], arows.at[gslot],
                                  semg.at[gslot]).wait()
            pltpu.make_async_copy(ctab.at[pl.ds(0, REB)], crows.at[gslot],
                                  semc.at[gslot]).wait()

        @pl.when(nbk > 0)
        def _pro():
            fire_g(0, 0)

        zf = jnp.zeros((16,), jnp.float32)
        ngv = jnp.full((16,), _NEG, jnp.float32)
        pzv = jnp.full((16,), _POS, jnp.float32)
        init = (zf, zf, zf, zf, ngv, ngv, pzv, pzv, zf, zf, NPT)

        def flush(sS0, sS1, sQ0, sQ1, sX0, sX1, sN0, sN1, dlc):
            accS[dlc, pl.ds(0, 16)] = sS0
            accS[dlc, pl.ds(16, 16)] = sS1
            accQ[dlc, pl.ds(0, 16)] = sQ0
            accQ[dlc, pl.ds(16, 16)] = sQ1
            accX[dlc, pl.ds(0, 16)] = sX0
            accX[dlc, pl.ds(16, 16)] = sX1
            accN[dlc, pl.ds(0, 16)] = sN0
            accN[dlc, pl.ds(16, 16)] = sN1

        def blk_body(b, carry):
            gslot = b & 1

            @pl.when(b + 1 < nbk)
            def _pf():
                fire_g(b + 1, (b + 1) & 1)

            wait_g(gslot)

            def grp_body(g, c2):
                off = pl.ds(pl.multiple_of(b * REB + g * 16, 16), 16)
                pkv = srcS[off]
                dlv = pkv >> 14
                cwv = cwS[off]
                tv = (cwv >> 14).astype(jnp.float32) * (1.0 / 511.0)
                (sS0, sS1, sQ0, sQ1, sX0, sX1, sN0, sN1, b0r, b1r, dlc) = c2

                for l in range(16):
                    dl = dlv[l]
                    t = tv[l]
                    changed = dl != dlc

                    def fl(sS0, sS1, sQ0, sQ1, sX0, sX1, sN0, sN1,
                           b0r, b1r, dlc, dl=dl):
                        flush(sS0, sS1, sQ0, sQ1, sX0, sX1, sN0, sN1, dlc)
                        nb0 = bvc[dl, pl.ds(0, 16)]
                        nb1 = bvc[dl, pl.ds(16, 16)]
                        return (zf, zf, zf, zf, ngv, ngv, pzv, pzv,
                                nb0, nb1, dl)

                    def kp(sS0, sS1, sQ0, sQ1, sX0, sX1, sN0, sN1,
                           b0r, b1r, dlc):
                        return (sS0, sS1, sQ0, sQ1, sX0, sX1, sN0, sN1,
                                b0r, b1r, dlc)

                    (sS0, sS1, sQ0, sQ1, sX0, sX1, sN0, sN1, b0r, b1r,
                     dlc) = lax.cond(changed, fl, kp, sS0, sS1, sQ0, sQ1,
                                     sX0, sX1, sN0, sN1, b0r, b1r, dlc)

                    e = g * 16 + l
                    a0 = arows[gslot, e, pl.ds(0, 16)]
                    a1 = arows[gslot, e, pl.ds(16, 16)]
                    cx0 = crows[gslot, e, pl.ds(0, 16)]
                    cx1 = crows[gslot, e, pl.ds(16, 16)]
                    c0e = plsc.bitcast(cx0 << 16, jnp.float32)
                    c0o = plsc.bitcast(cx0 & (-65536), jnp.float32)
                    c1e = plsc.bitcast(cx1 << 16, jnp.float32)
                    c1o = plsc.bitcast(cx1 & (-65536), jnp.float32)
                    m0 = a0 + b0r + c0e + t * (c1e - c0e)
                    m1 = a1 + b1r + c0o + t * (c1o - c0o)
                    m0 = jnp.maximum(m0, 0.0)
                    m1 = jnp.maximum(m1, 0.0)
                    sS0 = sS0 + m0
                    sS1 = sS1 + m1
                    sQ0 = sQ0 + m0 * m0
                    sQ1 = sQ1 + m1 * m1
                    sX0 = jnp.maximum(sX0, m0)
                    sX1 = jnp.maximum(sX1, m1)
                    sN0 = jnp.minimum(sN0, m0)
                    sN1 = jnp.minimum(sN1, m1)

                return (sS0, sS1, sQ0, sQ1, sX0, sX1, sN0, sN1, b0r, b1r, dlc)

            return lax.fori_loop(0, REB // 16, grp_body, carry)

        fin = lax.fori_loop(0, nbk, blk_body, init)
        (sS0, sS1, sQ0, sQ1, sX0, sX1, sN0, sN1, _, _, dlc) = fin
        flush(sS0, sS1, sQ0, sQ1, sX0, sX1, sN0, sN1, dlc)

        pltpu.sync_copy(accS.at[pl.ds(0, NPT)], sum_h.at[p, pl.ds(node0, NPT)])
        pltpu.sync_copy(accQ.at[pl.ds(0, NPT)], sq_h.at[p, pl.ds(node0, NPT)])
        pltpu.sync_copy(accX.at[pl.ds(0, NPT)], mx_h.at[p, pl.ds(node0, NPT)])
        pltpu.sync_copy(accN.at[pl.ds(0, NPT)], mn_h.at[p, pl.ds(node0, NPT)])
        return 0

    lax.fori_loop(0, NCH, pass_body, 0)
    pltpu.sync_copy(startv.at[pl.ds(0, 328)],
                    startv_h.at[pl.ds(pl.multiple_of(wid * 328, 8), 328)])


def _head_body(feat_ref, w1_ref, b1_ref, w2_ref, b2_ref, w3_ref, b3_ref, out_ref):
    h = jnp.maximum(feat_ref[...] @ w1_ref[...] + b1_ref[...], 0.0)
    h = jnp.maximum(h @ w2_ref[...] + b2_ref[...], 0.0)
    out_ref[...] = h @ w3_ref[...] + b3_ref[...]


def _ln(x, eps=1e-5):
    m = jnp.mean(x, axis=-1, keepdims=True)
    v = jnp.var(x, axis=-1, keepdims=True)
    return (x - m) / jnp.sqrt(v + eps)


def _seg_max(x, ids, n):
    m = jax.ops.segment_max(x, ids, num_segments=n)
    return jnp.where(jnp.isfinite(m), m, 0.0)


def kernel(mol_x, mol_x_feat, mol_total_fea, residue_esm, residue_prot5, residue_edge_index, residue_edge_weight, mol_batch, prot_batch, W_esm, b_esm, W_prot5, b_prot5, W_seq, b_seq, emb_atom, W_af, b_af, W_mol, b_mol, W_mol2, b_mol2, W_msg, b_msg, W_post, b_post, W_c1, b_c1, W_c2, b_c2, W_c3, b_c3):
    relu = jax.nn.relu
    B = mol_total_fea.shape[0]

    residue_ini = jnp.concatenate(
        [relu(residue_prot5 @ W_prot5 + b_prot5), relu(residue_esm @ W_esm + b_esm)], axis=-1)
    residue_x = relu(residue_ini @ W_seq + b_seq)

    W1 = W_msg[0:H]
    W2 = W_msg[H:2 * H]
    W3 = W_msg[2 * H:3 * H]
    A = residue_x @ W1
    Bv = residue_x @ W2 + b_msg

    perm_l = []
    for c in range(NCH):
        perm_l += [c * F + j for j in range(0, F, 2)]
        perm_l += [c * F + j for j in range(1, F, 2)]
    perm = jnp.array(perm_l, jnp.int32)
    inv = jnp.array([perm_l.index(i) for i in range(HP)], jnp.int32)

    A_p = jnp.zeros((NP, HP), jnp.float32).at[:N, :H].set(A)[:, perm]
    A_st = A_p.reshape(NP, NCH, F).transpose(1, 0, 2).reshape(NCH * NP, F)
    Bv_p = jnp.zeros((NP, HP), jnp.float32).at[:N, :H].set(Bv)[:, perm]
    Bv_st = Bv_p.reshape(NP, NCH, F).transpose(1, 0, 2)

    grid = jnp.arange(KG + 1, dtype=jnp.float32) / KG
    mu = jnp.linspace(0.0, 1.0, H)
    rbf_g = jnp.exp(-(((grid[:, None] - mu[None, :]) * H) ** 2))
    C_full = jnp.zeros((KG + 1, HP), jnp.float32).at[:, :H].set(rbf_g @ W3)
    cb = jax.lax.bitcast_convert_type(
        C_full.astype(jnp.bfloat16), jnp.uint16).astype(jnp.uint32)
    cpk = jax.lax.bitcast_convert_type(
        cb[:, 0::2] | (cb[:, 1::2] << 16), jnp.int32)
    P0 = cpk[:KG].reshape(KG, NCH, F // 2)
    P1 = cpk[1:].reshape(KG, NCH, F // 2)
    ctab = jnp.concatenate([P0, P1], axis=-1).transpose(1, 0, 2).reshape(NCH * KG, F)

    src = residue_edge_index[0].astype(jnp.int32)
    dst = residue_edge_index[1].astype(jnp.int32)
    w = residue_edge_weight.astype(jnp.float32)

    src_b, dst_b, w_b, hist = _bucket_kernel(src, dst, w)
    histt = jnp.broadcast_to(
        hist.reshape(NT, NT).T[:, :, None], (NT, NT, 8)).reshape(-1)
    sum_h, sq_h, mx_h, mn_h, startv_h = _edge_kernel(
        A_st, Bv_st, ctab, src_b, dst_b, w_b, histt)

    def unstack(x):
        return x.transpose(1, 0, 2).reshape(NP, HP)[:, inv][:N, :H]

    sh = startv_h.reshape(NT, 328)
    cnt = (sh[:, 1:NPT + 1] - sh[:, :NPT]).reshape(NP)[:N].astype(jnp.float32)
    s = unstack(sum_h)
    q = unstack(sq_h)
    mx = unstack(mx_h)
    mn = unstack(mn_h)
    has = (cnt > 0.5)[:, None]
    cntc = jnp.maximum(cnt, 1.0)[:, None]
    mean = jnp.where(has, s / cntc, 0.0)
    sq = jnp.where(has, q / cntc, 0.0)
    std = jnp.sqrt(relu(sq - mean * mean) + 1e-5)
    mx = jnp.where(has, mx, 0.0)
    mn = jnp.where(has, mn, 0.0)
    agg = jnp.concatenate([mean, mn, mx, std], axis=-1)
    residue_x2 = relu(agg @ W_post + b_post)

    atom_oh = (mol_x[:, None] == jnp.arange(emb_atom.shape[0])[None, :]).astype(jnp.float32)
    atom_x = atom_oh @ emb_atom + relu(mol_x_feat @ W_af + b_af)
    mol_total = _ln(relu(mol_total_fea @ W_mol + b_mol) @ W_mol2 + b_mol2)

    prot_oh = (prot_batch[:, None] == jnp.arange(B)[None, :]).astype(jnp.float32)
    pc0 = prot_oh.sum(axis=0)
    residue_mean = (prot_oh.T @ residue_x2) / jnp.maximum(pc0, 1.0)[:, None]
    residue_max = _seg_max(residue_x2, prot_batch, B)
    atom_pool = _seg_max(atom_x, mol_batch, B)
    feat = jnp.concatenate([residue_max, residue_mean, atom_pool, mol_total], axis=-1)

    out = pl.pallas_call(
        _head_body,
        out_shape=jax.ShapeDtypeStruct((B, 1), jnp.float32),
    )(feat, W_c1, b_c1, W_c2, b_c2, W_c3, b_c3)
    return out
